# baseline (device time: 53079 ns/iter reference)
import jax
import jax.numpy as jnp
from jax import lax
from jax.experimental import pallas as pl
from jax.experimental.pallas import tpu as pltpu

N_DEV = 16
H = 8
B = 3
S = 2

RING = [0, 1, 5, 9, 13, 14, 10, 6, 2, 3, 7, 11, 15, 12, 8, 4]
INV = [RING.index(p) for p in range(N_DEV)]


def kernel(x, w_mat):
    m_per, k = x.shape
    _, n_per = w_mat.shape

    def body(
        x_ref, w_ref, out_ref,
        x_bf, comm_r, comm_l, w_bf_ref,
        send_r, recv_r, send_l, recv_l,
    ):
        my_pos = lax.axis_index("i")

        def table(idx, vals):
            out = jnp.int32(vals[0])
            for j in range(1, len(vals)):
                out = jnp.where(idx == j, jnp.int32(vals[j]), out)
            return out

        ridx = table(my_pos, INV)
        right = table((ridx + 1) % N_DEV, RING)
        left = table((ridx + N_DEV - 1) % N_DEV, RING)

        def origin_r(h):
            return table((ridx + N_DEV - h - 1) % N_DEV, RING)

        def origin_l(h):
            return table((ridx + h + 1) % N_DEV, RING)

        barrier_sem = pltpu.get_barrier_semaphore()
        for nbr in [left, right]:
            pl.semaphore_signal(
                barrier_sem, inc=1,
                device_id=(nbr,), device_id_type=pl.DeviceIdType.MESH,
            )
        x_bf[...] = x_ref[...].astype(jnp.bfloat16)
        pl.semaphore_wait(barrier_sem, 2)

        def gemm_store(origin, chunk, row_off, nrows):
            acc = jnp.dot(chunk, w_bf_ref[...], preferred_element_type=jnp.float32)
            out_ref[pl.ds(origin * m_per + row_off, nrows), :] = acc * (
                1.0 / (1.0 + jnp.exp(-acc))
            )

        sub_m = m_per // S

        def rdma_hop(comm, send_sems, recv_sems, h, s, target):
            rows = pl.ds(s * sub_m, sub_m)
            src = x_bf.at[rows] if h == 0 else comm.at[(h - 1) % B, rows]
            return pltpu.make_async_remote_copy(
                src_ref=src,
                dst_ref=comm.at[h % B, rows],
                send_sem=send_sems.at[h % B, s],
                recv_sem=recv_sems.at[h % B, s],
                device_id=(target,),
                device_id_type=pl.DeviceIdType.MESH,
            )

        def subs_r(h):
            return range(S) if h < H - 1 else (0,)

        def subs_l(h):
            return range(S) if h < H - 1 else (1,)

        for s in range(S):
            rdma_hop(comm_r, send_r, recv_r, 0, s, right).start()
            rdma_hop(comm_l, send_l, recv_l, 0, s, left).start()
        w_bf_ref[...] = w_ref[...].astype(jnp.bfloat16)
        gemm_store(my_pos, x_bf[...], 0, m_per)

        for h in range(H):
            for s in range(S):
                if s in subs_r(h):
                    rr = rdma_hop(comm_r, send_r, recv_r, h, s, right)
                    rr.wait_recv()
                    rr.wait_send()
                    if h + 1 < H and s in subs_r(h + 1):
                        rdma_hop(comm_r, send_r, recv_r, h + 1, s, right).start()
                if s in subs_l(h):
                    rl = rdma_hop(comm_l, send_l, recv_l, h, s, left)
                    rl.wait_recv()
                    rl.wait_send()
                    if h + 1 < H and s in subs_l(h + 1):
                        rdma_hop(comm_l, send_l, recv_l, h + 1, s, left).start()
            if h < H - 1:
                gemm_store(origin_r(h), comm_r[h % B], 0, m_per)
                gemm_store(origin_l(h), comm_l[h % B], 0, m_per)
            else:
                half = m_per // 2
                gemm_store(origin_r(h), comm_r[h % B, :half], 0, half)
                gemm_store(origin_l(h), comm_l[h % B, half:], half, half)

    return pl.pallas_call(
        body,
        out_shape=jax.ShapeDtypeStruct((N_DEV * m_per, n_per), jnp.float32),
        in_specs=[
            pl.BlockSpec(memory_space=pltpu.VMEM),
            pl.BlockSpec(memory_space=pltpu.VMEM),
        ],
        out_specs=pl.BlockSpec(memory_space=pltpu.VMEM),
        scratch_shapes=[
            pltpu.VMEM((m_per, k), jnp.bfloat16),
            pltpu.VMEM((B, m_per, k), jnp.bfloat16),
            pltpu.VMEM((B, m_per, k), jnp.bfloat16),
            pltpu.VMEM((k, n_per), jnp.bfloat16),
            pltpu.SemaphoreType.DMA((B, S)),
            pltpu.SemaphoreType.DMA((B, S)),
            pltpu.SemaphoreType.DMA((B, S)),
            pltpu.SemaphoreType.DMA((B, S)),
        ],
        compiler_params=pltpu.CompilerParams(collective_id=0),
    )(x, w_mat)


# device time: 51657 ns/iter; 1.0275x vs baseline; 1.0275x over previous
import jax
import jax.numpy as jnp
from jax import lax
from jax.experimental import pallas as pl
from jax.experimental.pallas import tpu as pltpu

N_DEV = 16
H = 8
B = 3
S = 8

RING = [0, 1, 5, 9, 13, 14, 10, 6, 2, 3, 7, 11, 15, 12, 8, 4]
INV = [RING.index(p) for p in range(N_DEV)]


def kernel(x, w_mat):
    m_per, k = x.shape
    _, n_per = w_mat.shape

    def body(
        x_ref, w_ref, out_ref,
        x_bf, comm_r, comm_l, w_bf_ref,
        send_r, recv_r, send_l, recv_l,
    ):
        my_pos = lax.axis_index("i")

        def table(idx, vals):
            out = jnp.int32(vals[0])
            for j in range(1, len(vals)):
                out = jnp.where(idx == j, jnp.int32(vals[j]), out)
            return out

        ridx = table(my_pos, INV)
        right = table((ridx + 1) % N_DEV, RING)
        left = table((ridx + N_DEV - 1) % N_DEV, RING)

        def origin_r(h):
            return table((ridx + N_DEV - h - 1) % N_DEV, RING)

        def origin_l(h):
            return table((ridx + h + 1) % N_DEV, RING)

        barrier_sem = pltpu.get_barrier_semaphore()
        for nbr in [left, right]:
            pl.semaphore_signal(
                barrier_sem, inc=1,
                device_id=(nbr,), device_id_type=pl.DeviceIdType.MESH,
            )
        x_bf[...] = x_ref[...].astype(jnp.bfloat16)
        pl.semaphore_wait(barrier_sem, 2)

        def gemm_store(origin, chunk, row_off, nrows):
            acc = jnp.dot(chunk, w_bf_ref[...], preferred_element_type=jnp.float32)
            out_ref[pl.ds(origin * m_per + row_off, nrows), :] = acc * (
                1.0 / (1.0 + jnp.exp(-acc))
            )

        sub_m = m_per // S

        def rdma_hop(comm, send_sems, recv_sems, h, s, target):
            rows = pl.ds(s * sub_m, sub_m)
            src = x_bf.at[rows] if h == 0 else comm.at[(h - 1) % B, rows]
            return pltpu.make_async_remote_copy(
                src_ref=src,
                dst_ref=comm.at[h % B, rows],
                send_sem=send_sems.at[h % B, s],
                recv_sem=recv_sems.at[h % B, s],
                device_id=(target,),
                device_id_type=pl.DeviceIdType.MESH,
            )

        def subs_r(h):
            return range(S) if h < H - 1 else (0, 1, 2, 3)

        def subs_l(h):
            return range(S) if h < H - 1 else (4, 5, 6, 7)

        for s in range(S):
            rdma_hop(comm_r, send_r, recv_r, 0, s, right).start()
            rdma_hop(comm_l, send_l, recv_l, 0, s, left).start()
        w_bf_ref[...] = w_ref[...].astype(jnp.bfloat16)
        gemm_store(my_pos, x_bf[...], 0, m_per)

        for h in range(H):
            for s in range(S):
                if s in subs_r(h):
                    rr = rdma_hop(comm_r, send_r, recv_r, h, s, right)
                    rr.wait_recv()
                    rr.wait_send()
                    if h + 1 < H and s in subs_r(h + 1):
                        rdma_hop(comm_r, send_r, recv_r, h + 1, s, right).start()
                if s in subs_l(h):
                    rl = rdma_hop(comm_l, send_l, recv_l, h, s, left)
                    rl.wait_recv()
                    rl.wait_send()
                    if h + 1 < H and s in subs_l(h + 1):
                        rdma_hop(comm_l, send_l, recv_l, h + 1, s, left).start()
            if h < H - 1:
                gemm_store(origin_r(h), comm_r[h % B], 0, m_per)
                gemm_store(origin_l(h), comm_l[h % B], 0, m_per)
            else:
                half = m_per // 2
                gemm_store(origin_r(h), comm_r[h % B, :half], 0, half)
                gemm_store(origin_l(h), comm_l[h % B, half:], half, half)

    return pl.pallas_call(
        body,
        out_shape=jax.ShapeDtypeStruct((N_DEV * m_per, n_per), jnp.float32),
        in_specs=[
            pl.BlockSpec(memory_space=pltpu.VMEM),
            pl.BlockSpec(memory_space=pltpu.VMEM),
        ],
        out_specs=pl.BlockSpec(memory_space=pltpu.VMEM),
        scratch_shapes=[
            pltpu.VMEM((m_per, k), jnp.bfloat16),
            pltpu.VMEM((B, m_per, k), jnp.bfloat16),
            pltpu.VMEM((B, m_per, k), jnp.bfloat16),
            pltpu.VMEM((k, n_per), jnp.bfloat16),
            pltpu.SemaphoreType.DMA((B, S)),
            pltpu.SemaphoreType.DMA((B, S)),
            pltpu.SemaphoreType.DMA((B, S)),
            pltpu.SemaphoreType.DMA((B, S)),
        ],
        compiler_params=pltpu.CompilerParams(collective_id=0),
    )(x, w_mat)
